# baseline (speedup 1.0000x reference)
import functools

import jax
import jax.numpy as jnp
from jax import lax
from jax.experimental import pallas as pl
from jax.experimental.pallas import tpu as pltpu
from jax.experimental.pallas import tpu_sc as plsc

F = 26
B = 16384
V = 100000
D = 32
ROWS = V + 1
R_TOT = F * D

NC = 2
NS = 16
L = 16
NW = NC * NS
RPW = R_TOT // NW
CHUNK = 2048
NCHUNK = B // CHUNK


def _emb_body(idx_hbm, tab_hbm, tail_hbm, out_hbm, idx_v, slab_v, out_a,
              out_b, sem_s, sem_a, sem_b):
    wid = lax.axis_index("s") * NC + lax.axis_index("c")
    r0 = wid * RPW
    f0 = r0 // D
    n0 = jnp.minimum(RPW, (f0 + 1) * D - r0)

    def row_body(r, warm):
        copies = [
            pltpu.async_copy(tab_hbm.at[r].at[pl.ds(0, 65536)],
                             slab_v.at[pl.ds(0, 65536)], sem_s),
            pltpu.async_copy(tab_hbm.at[r].at[pl.ds(65536, 34432)],
                             slab_v.at[pl.ds(65536, 34432)], sem_s),
            pltpu.async_copy(tail_hbm.at[r],
                             slab_v.at[pl.ds(99968, 128)], sem_s),
        ]
        for cp in copies:
            cp.wait()

        def pair_body(j, w):
            for half, (buf, sem) in enumerate(
                ((out_a, sem_a), (out_b, sem_b))):
                coff = (2 * j + half) * CHUNK
                dst = out_hbm.at[r, pl.ds(coff, CHUNK)]

                @pl.when(w > 0)
                def _():
                    pltpu.make_async_copy(buf, dst, sem).wait()

                @plsc.parallel_loop(0, CHUNK // L, unroll=16)
                def _(k):
                    vi = idx_v[pl.ds(coff + k * L, L)]
                    buf[pl.ds(k * L, L)] = plsc.load_gather(slab_v, [vi + 1])
                pltpu.async_copy(buf, dst, sem)
            return jnp.int32(1)

        return lax.fori_loop(0, NCHUNK // 2, pair_body, warm)

    pltpu.sync_copy(idx_hbm.at[f0], idx_v)
    s1 = lax.fori_loop(r0, r0 + n0, row_body, jnp.int32(0))

    @pl.when(n0 < RPW)
    def _():
        pltpu.sync_copy(idx_hbm.at[f0 + 1], idx_v)

    lax.fori_loop(r0 + n0, r0 + RPW, row_body, s1)

    last = r0 + RPW - 1
    pltpu.make_async_copy(
        out_a, out_hbm.at[last, pl.ds((NCHUNK - 2) * CHUNK, CHUNK)], sem_a
    ).wait()
    pltpu.make_async_copy(
        out_b, out_hbm.at[last, pl.ds((NCHUNK - 1) * CHUNK, CHUNK)], sem_b
    ).wait()


def kernel(indices, tables):
    tab2 = jnp.transpose(tables, (0, 2, 1)).reshape(R_TOT, ROWS)
    mesh = plsc.VectorSubcoreMesh(core_axis_name="c", subcore_axis_name="s")
    emb = functools.partial(
        pl.kernel,
        mesh=mesh,
        compiler_params=pltpu.CompilerParams(needs_layout_passes=False),
        out_type=jax.ShapeDtypeStruct((R_TOT, B), jnp.float32),
        scratch_types=[
            pltpu.VMEM((B,), jnp.int32),
            pltpu.VMEM((100096,), jnp.float32),
            pltpu.VMEM((CHUNK,), jnp.float32),
            pltpu.VMEM((CHUNK,), jnp.float32),
            pltpu.SemaphoreType.DMA,
            pltpu.SemaphoreType.DMA,
            pltpu.SemaphoreType.DMA,
        ],
    )(_emb_body)
    tail = jnp.concatenate(
        [tab2[:, 99968:], jnp.zeros((R_TOT, 95), jnp.float32)], axis=1)
    out = emb(indices, tab2, tail)
    return out.T


# speedup vs baseline: 1.0024x; 1.0024x over previous
import functools

import jax
import jax.numpy as jnp
from jax import lax
from jax.experimental import pallas as pl
from jax.experimental.pallas import tpu as pltpu
from jax.experimental.pallas import tpu_sc as plsc

F = 26
B = 16384
V = 100000
D = 32
ROWS = V + 1
R_TOT = F * D

NC = 2
NS = 16
L = 16
NW = NC * NS
RPW = R_TOT // NW
CHUNK = 2048
NCHUNK = B // CHUNK


def _emb_body(idx_hbm, tab_hbm, out_hbm, idx_v, slab_v, out_a, out_b, sem_s,
              sem_a, sem_b):
    wid = lax.axis_index("s") * NC + lax.axis_index("c")
    r0 = wid * RPW
    f0 = r0 // D
    n0 = jnp.minimum(RPW, (f0 + 1) * D - r0)

    def row_body(r, warm):
        pltpu.async_copy(tab_hbm.at[r], slab_v, sem_s).wait()

        def pair_body(j, w):
            for half, (buf, sem) in enumerate(
                ((out_a, sem_a), (out_b, sem_b))):
                coff = (2 * j + half) * CHUNK
                dst = out_hbm.at[r, pl.ds(coff, CHUNK)]

                @pl.when(w > 0)
                def _():
                    pltpu.make_async_copy(buf, dst, sem).wait()

                @plsc.parallel_loop(0, CHUNK // L, unroll=8)
                def _(k):
                    vi = idx_v[pl.ds(coff + k * L, L)]
                    buf[pl.ds(k * L, L)] = plsc.load_gather(slab_v, [vi + 1])
                pltpu.async_copy(buf, dst, sem)
            return jnp.int32(1)

        return lax.fori_loop(0, NCHUNK // 2, pair_body, warm)

    pltpu.sync_copy(idx_hbm.at[f0], idx_v)
    s1 = lax.fori_loop(r0, r0 + n0, row_body, jnp.int32(0))

    @pl.when(n0 < RPW)
    def _():
        pltpu.sync_copy(idx_hbm.at[f0 + 1], idx_v)

    lax.fori_loop(r0 + n0, r0 + RPW, row_body, s1)

    last = r0 + RPW - 1
    pltpu.make_async_copy(
        out_a, out_hbm.at[last, pl.ds((NCHUNK - 2) * CHUNK, CHUNK)], sem_a
    ).wait()
    pltpu.make_async_copy(
        out_b, out_hbm.at[last, pl.ds((NCHUNK - 1) * CHUNK, CHUNK)], sem_b
    ).wait()


def kernel(indices, tables):
    tab2 = jnp.transpose(tables, (0, 2, 1)).reshape(R_TOT, ROWS)
    mesh = plsc.VectorSubcoreMesh(core_axis_name="c", subcore_axis_name="s")
    emb = functools.partial(
        pl.kernel,
        mesh=mesh,
        compiler_params=pltpu.CompilerParams(needs_layout_passes=False),
        out_type=jax.ShapeDtypeStruct((R_TOT, B), jnp.float32),
        scratch_types=[
            pltpu.VMEM((B,), jnp.int32),
            pltpu.VMEM((ROWS,), jnp.float32),
            pltpu.VMEM((CHUNK,), jnp.float32),
            pltpu.VMEM((CHUNK,), jnp.float32),
            pltpu.SemaphoreType.DMA,
            pltpu.SemaphoreType.DMA,
            pltpu.SemaphoreType.DMA,
        ],
    )(_emb_body)
    out = emb(indices, tab2)
    return out.T


# speedup vs baseline: 1.0057x; 1.0032x over previous
import functools

import jax
import jax.numpy as jnp
from jax import lax
from jax.experimental import pallas as pl
from jax.experimental.pallas import tpu as pltpu
from jax.experimental.pallas import tpu_sc as plsc

F = 26
B = 16384
V = 100000
D = 32
ROWS = V + 1
R_TOT = F * D

NC = 2
NS = 16
L = 16
NW = NC * NS
RPW = R_TOT // NW
CHUNK = 4096
NCHUNK = B // CHUNK


def _emb_body(idx_hbm, tab_hbm, out_hbm, idx_v, slab_v, out_a, out_b, sem_s,
              sem_a, sem_b):
    wid = lax.axis_index("s") * NC + lax.axis_index("c")
    r0 = wid * RPW
    f0 = r0 // D
    n0 = jnp.minimum(RPW, (f0 + 1) * D - r0)

    def row_body(r, warm):
        pltpu.async_copy(tab_hbm.at[r], slab_v, sem_s).wait()

        def pair_body(j, w):
            for half, (buf, sem) in enumerate(
                ((out_a, sem_a), (out_b, sem_b))):
                coff = (2 * j + half) * CHUNK
                dst = out_hbm.at[r, pl.ds(coff, CHUNK)]

                @pl.when(w > 0)
                def _():
                    pltpu.make_async_copy(buf, dst, sem).wait()

                @plsc.parallel_loop(0, CHUNK // L, unroll=8)
                def _(k):
                    vi = idx_v[pl.ds(coff + k * L, L)]
                    buf[pl.ds(k * L, L)] = plsc.load_gather(slab_v, [vi + 1])
                pltpu.async_copy(buf, dst, sem)
            return jnp.int32(1)

        return lax.fori_loop(0, NCHUNK // 2, pair_body, warm)

    pltpu.sync_copy(idx_hbm.at[f0], idx_v)
    s1 = lax.fori_loop(r0, r0 + n0, row_body, jnp.int32(0))

    @pl.when(n0 < RPW)
    def _():
        pltpu.sync_copy(idx_hbm.at[f0 + 1], idx_v)

    lax.fori_loop(r0 + n0, r0 + RPW, row_body, s1)

    last = r0 + RPW - 1
    pltpu.make_async_copy(
        out_a, out_hbm.at[last, pl.ds((NCHUNK - 2) * CHUNK, CHUNK)], sem_a
    ).wait()
    pltpu.make_async_copy(
        out_b, out_hbm.at[last, pl.ds((NCHUNK - 1) * CHUNK, CHUNK)], sem_b
    ).wait()


def kernel(indices, tables):
    tab2 = jnp.transpose(tables, (0, 2, 1)).reshape(R_TOT, ROWS)
    mesh = plsc.VectorSubcoreMesh(core_axis_name="c", subcore_axis_name="s")
    emb = functools.partial(
        pl.kernel,
        mesh=mesh,
        compiler_params=pltpu.CompilerParams(needs_layout_passes=False),
        out_type=jax.ShapeDtypeStruct((R_TOT, B), jnp.float32),
        scratch_types=[
            pltpu.VMEM((B,), jnp.int32),
            pltpu.VMEM((ROWS,), jnp.float32),
            pltpu.VMEM((CHUNK,), jnp.float32),
            pltpu.VMEM((CHUNK,), jnp.float32),
            pltpu.SemaphoreType.DMA,
            pltpu.SemaphoreType.DMA,
            pltpu.SemaphoreType.DMA,
        ],
    )(_emb_body)
    out = emb(indices, tab2)
    return out.T
